# baseline (speedup 1.0000x reference)
import jax
import jax.numpy as jnp
from jax import lax
from jax.experimental import pallas as pl
from jax.experimental.pallas import tpu as pltpu
from jax.experimental.pallas import tpu_sc as plsc

N_NODES = 10000
E_EDGES = 320000
D = 128
D_EDGE = 16

NC = 2
NS = 16
NW = NC * NS

LANES = 16
SEG = 2048
E_PAD = 327680
NSEG = E_PAD // SEG
PAD_DST = 2 ** 30
N_ACC = NW * 320
OWN = N_ACC // NW
BLK = 128
LC = SEG + BLK


def _sc_body(x_a_h, x_b_h,
             src_ab_h, dst_ab_h, attr_ab_h,
             src_ba_h, dst_ba_h, attr_ba_h,
             s_ab_o, t_ab_o, c_ab_o, s_ba_o, t_ba_o, c_ba_o,
             S_acc, T_acc, cnt_v, srcb, dstb, src_l, loc_l, eid_l,
             xbuf, abuf, gsem, asem):
  c = lax.axis_index("c")
  s = lax.axis_index("s")
  wid = s * NC + c
  lo = wid * OWN

  zf = jnp.zeros((LANES,), jnp.float32)
  zi = jnp.zeros((LANES,), jnp.int32)
  ones_i = jnp.full((LANES,), 1, jnp.int32)
  iota16 = lax.iota(jnp.int32, LANES)

  def _run_relation(x_h, src_h, dst_h, attr_h, s_o, t_o, c_o):
    def _zs(i, _):
      S_acc[i // 8, pl.ds((i % 8) * LANES, LANES)] = zf
      return 0
    lax.fori_loop(0, (OWN + 8) * 8, _zs, 0)

    def _zt(i, _):
      T_acc[i, :] = zf
      return 0
    lax.fori_loop(0, OWN + 8, _zt, 0)

    def _zc(i, _):
      cnt_v[pl.ds(i * LANES, LANES)] = zi
      return 0
    lax.fori_loop(0, (OWN + LANES) // LANES, _zc, 0)

    def _drain_block(b0):
      pltpu.async_copy(x_h.at[src_l.at[pl.ds(b0, BLK)]], xbuf, gsem)
      pltpu.async_copy(attr_h.at[eid_l.at[pl.ds(b0, BLK)]], abuf, asem)
      pltpu.make_async_copy(x_h.at[src_l.at[pl.ds(b0, BLK)]], xbuf, gsem).wait()
      pltpu.make_async_copy(attr_h.at[eid_l.at[pl.ds(b0, BLK)]], abuf,
                            asem).wait()

      def _acc16(g, _):
        lv = loc_l[pl.ds(b0 + g * LANES, LANES)]
        for j in range(LANES):
          loc = lv[j]
          row = g * LANES + j
          for q in range(D // LANES):
            plsc.addupdate(S_acc.at[loc, pl.ds(q * LANES, LANES)],
                           xbuf[row, pl.ds(q * LANES, LANES)])
          plsc.addupdate(T_acc.at[loc], abuf[row])
        return 0
      lax.fori_loop(0, BLK // LANES, _acc16, 0)

      def _cix(q, _):
        lv = loc_l[pl.ds(b0 + q * LANES, LANES)]
        plsc.addupdate_scatter(cnt_v, [lv], ones_i)
        return 0
      lax.fori_loop(0, BLK // LANES, _cix, 0)

    def _chunk(ci, off):
      e0 = ci * SEG
      pltpu.sync_copy(src_h.at[pl.ds(e0, SEG)], srcb)
      pltpu.sync_copy(dst_h.at[pl.ds(e0, SEG)], dstb)

      def _scan(k, off):
        dstv = dstb[pl.ds(k * LANES, LANES)]
        srcv = srcb[pl.ds(k * LANES, LANES)]
        locv = dstv - lo
        m = jnp.logical_and(locv >= 0, locv < OWN)
        eidv = iota16 + (e0 + k * LANES)
        plsc.store_compressed(loc_l.at[pl.ds(off, LANES)], locv, mask=m)
        plsc.store_compressed(src_l.at[pl.ds(off, LANES)], srcv, mask=m)
        plsc.store_compressed(eid_l.at[pl.ds(off, LANES)], eidv, mask=m)
        return off + jnp.sum(m.astype(jnp.int32))
      off = lax.fori_loop(0, SEG // LANES, _scan, off)

      nblk = off // BLK

      def _blk(i, _):
        _drain_block(i * BLK)
        return 0
      lax.fori_loop(0, nblk, _blk, 0)

      r0 = nblk * BLK
      for q in range(BLK // LANES):
        sv = src_l[pl.ds(r0 + q * LANES, LANES)]
        lv = loc_l[pl.ds(r0 + q * LANES, LANES)]
        ev = eid_l[pl.ds(r0 + q * LANES, LANES)]
        src_l[pl.ds(q * LANES, LANES)] = sv
        loc_l[pl.ds(q * LANES, LANES)] = lv
        eid_l[pl.ds(q * LANES, LANES)] = ev
      return off - nblk * BLK
    off = lax.fori_loop(0, NSEG, _chunk, 0)

    dumpv = jnp.full((LANES,), OWN, jnp.int32)
    for q in range(BLK // LANES):
      src_l[pl.ds(off + q * LANES, LANES)] = zi
      loc_l[pl.ds(off + q * LANES, LANES)] = dumpv
      eid_l[pl.ds(off + q * LANES, LANES)] = zi
    _drain_block(0)

    pltpu.sync_copy(S_acc.at[pl.ds(0, OWN)], s_o.at[pl.ds(lo, OWN)])
    pltpu.sync_copy(T_acc.at[pl.ds(0, OWN)], t_o.at[pl.ds(lo, OWN)])
    pltpu.sync_copy(cnt_v.at[pl.ds(0, OWN)], c_o.at[0, pl.ds(lo, OWN)])

  _run_relation(x_a_h, src_ab_h, dst_ab_h, attr_ab_h, s_ab_o, t_ab_o, c_ab_o)
  _run_relation(x_b_h, src_ba_h, dst_ba_h, attr_ba_h, s_ba_o, t_ba_o, c_ba_o)


def _sc_segsums(x_a, x_b, src_ab, dst_ab, attr_ab, src_ba, dst_ba, attr_ba):
  mesh = plsc.VectorSubcoreMesh(core_axis_name="c", subcore_axis_name="s")
  f32 = jnp.float32
  out_type = (
      jax.ShapeDtypeStruct((N_ACC, D), f32),
      jax.ShapeDtypeStruct((N_ACC, D_EDGE), f32),
      jax.ShapeDtypeStruct((1, N_ACC), jnp.int32),
      jax.ShapeDtypeStruct((N_ACC, D), f32),
      jax.ShapeDtypeStruct((N_ACC, D_EDGE), f32),
      jax.ShapeDtypeStruct((1, N_ACC), jnp.int32),
  )
  scratch = [
      pltpu.VMEM((OWN + 8, D), f32),
      pltpu.VMEM((OWN + 8, D_EDGE), f32),
      pltpu.VMEM((OWN + LANES,), jnp.int32),
      pltpu.VMEM((SEG,), jnp.int32),
      pltpu.VMEM((SEG,), jnp.int32),
      pltpu.VMEM((LC,), jnp.int32),
      pltpu.VMEM((LC,), jnp.int32),
      pltpu.VMEM((LC,), jnp.int32),
      pltpu.VMEM((BLK, D), f32),
      pltpu.VMEM((BLK, D_EDGE), f32),
      pltpu.SemaphoreType.DMA,
      pltpu.SemaphoreType.DMA,
  ]
  return pl.kernel(
      _sc_body, out_type=out_type, mesh=mesh, scratch_types=scratch,
      compiler_params=pltpu.CompilerParams(
          needs_layout_passes=False, use_tc_tiling_on_sc=False),
  )(x_a, x_b, src_ab, dst_ab, attr_ab, src_ba, dst_ba, attr_ba)


BN = 2000


def _dense_body(x_ref, s_ref, t_ref, c_ref,
                wn_top, wn_bot, bn_r, ws_r, bs_r, wu_top, wu_bot, bu_r,
                wsf_r, bsf_r, out_ref):
  hi = jax.lax.Precision.HIGHEST
  x = x_ref[...]
  S = s_ref[...]
  T = t_ref[...]
  cnt = c_ref[...].astype(jnp.float32)
  summed = (jnp.dot(S, wn_top[...], precision=hi)
            + jnp.dot(T, wn_bot[...], precision=hi)
            + cnt * bn_r[...])
  agg = summed / jnp.maximum(cnt, 1.0)
  self_t = jnp.dot(x, ws_r[...], precision=hi) + bs_r[...]
  m = (jnp.dot(self_t, wu_top[...], precision=hi)
       + jnp.dot(agg, wu_bot[...], precision=hi) + bu_r[...])
  out_ref[...] = m + jnp.dot(x, wsf_r[...], precision=hi) + bsf_r[...]


def _dense(x, s_full, t_full, c_full, wn_top, wn_bot, bn, ws, bs,
           wu_top, wu_bot, bu, wsf, bsf):
  n = x.shape[0]
  grid = (n // BN,)
  row_spec = lambda width: pl.BlockSpec((BN, width), lambda i: (i, 0))
  full = lambda a: pl.BlockSpec(a.shape, lambda i: (0,) * a.ndim)
  return pl.pallas_call(
      _dense_body,
      grid=grid,
      in_specs=[
          row_spec(D), row_spec(D), row_spec(D_EDGE),
          pl.BlockSpec((BN, 1), lambda i: (i, 0)),
          full(wn_top), full(wn_bot), full(bn), full(ws), full(bs),
          full(wu_top), full(wu_bot), full(bu), full(wsf), full(bsf),
      ],
      out_specs=row_spec(D),
      out_shape=jax.ShapeDtypeStruct((n, D), jnp.float32),
  )(x, s_full, t_full, c_full,
    wn_top, wn_bot, bn, ws, bs, wu_top, wu_bot, bu, wsf, bsf)


def _pad_edges(edge_index, edge_attr):
  src = edge_index[0]
  dst = edge_index[1]
  pad = E_PAD - E_EDGES
  src = jnp.concatenate([src, jnp.zeros((pad,), jnp.int32)])
  dst = jnp.concatenate([dst, jnp.full((pad,), PAD_DST, jnp.int32)])
  attr = jnp.concatenate([edge_attr, jnp.zeros((pad, D_EDGE), jnp.float32)])
  return src, dst, attr


@jax.jit
def kernel(x_a, x_b, edge_index_ab, edge_index_ba, edge_attr_ab, edge_attr_ba,
           W_neigh_ab, b_neigh_ab, W_self_ab, b_self_ab, W_update_ab, b_update_ab,
           W_neigh_ba, b_neigh_ba, W_self_ba, b_self_ba, W_update_ba, b_update_ba,
           W_sf_a, b_sf_a, W_sf_b, b_sf_b):
  src_ab, dst_ab, attr_ab = _pad_edges(edge_index_ab, edge_attr_ab)
  src_ba, dst_ba, attr_ba = _pad_edges(edge_index_ba, edge_attr_ba)

  s_ab, t_ab, c_ab, s_ba, t_ba, c_ba = _sc_segsums(
      x_a, x_b, src_ab, dst_ab, attr_ab, src_ba, dst_ba, attr_ba)

  def two_d(b):
    return b.reshape(1, D)

  out_a = _dense(x_a, s_ba, t_ba, c_ba.reshape(N_ACC, 1),
                 W_neigh_ba[:D], W_neigh_ba[D:], two_d(b_neigh_ba),
                 W_self_ba, two_d(b_self_ba),
                 W_update_ba[:D], W_update_ba[D:], two_d(b_update_ba),
                 W_sf_a, two_d(b_sf_a))
  out_b = _dense(x_b, s_ab, t_ab, c_ab.reshape(N_ACC, 1),
                 W_neigh_ab[:D], W_neigh_ab[D:], two_d(b_neigh_ab),
                 W_self_ab, two_d(b_self_ab),
                 W_update_ab[:D], W_update_ab[D:], two_d(b_update_ab),
                 W_sf_b, two_d(b_sf_b))
  return (out_a, out_b)


# speedup vs baseline: 1.1973x; 1.1973x over previous
import jax
import jax.numpy as jnp
from jax import lax
from jax.experimental import pallas as pl
from jax.experimental.pallas import tpu as pltpu
from jax.experimental.pallas import tpu_sc as plsc

N_NODES = 10000
E_EDGES = 320000
D = 128
D_EDGE = 16

NC = 2
NS = 16
NW = NC * NS

LANES = 16
SEG = 2048
E_PAD = 327680
NSEG = E_PAD // SEG
PAD_DST = 2 ** 30
N_ACC = NW * 320
OWN = N_ACC // NW
BLK = 128
LC = SEG + BLK


def _sc_body(x_a_h, x_b_h,
             src_ab_h, dst_ab_h, attr_ab_h,
             src_ba_h, dst_ba_h, attr_ba_h,
             s_ab_o, t_ab_o, c_ab_o, s_ba_o, t_ba_o, c_ba_o,
             S_acc, T_acc, cnt_v, srcb, dstb, src_l, loc_l, eid_l,
             xbuf, abuf, gsem, asem):
  c = lax.axis_index("c")
  s = lax.axis_index("s")
  wid = s * NC + c
  lo = wid * OWN

  zf = jnp.zeros((LANES,), jnp.float32)
  zi = jnp.zeros((LANES,), jnp.int32)
  ones_i = jnp.full((LANES,), 1, jnp.int32)
  iota16 = lax.iota(jnp.int32, LANES)

  def _run_relation(x_h, src_h, dst_h, attr_h, s_o, t_o, c_o):
    def _zs(i, _):
      S_acc[i // 8, pl.ds((i % 8) * LANES, LANES)] = zf
      return 0
    lax.fori_loop(0, (OWN + 8) * 8, _zs, 0)

    def _zt(i, _):
      T_acc[i, :] = zf
      return 0
    lax.fori_loop(0, OWN + 8, _zt, 0)

    def _zc(i, _):
      cnt_v[pl.ds(i * LANES, LANES)] = zi
      return 0
    lax.fori_loop(0, (OWN + LANES) // LANES, _zc, 0)

    def _drain_block(b0):
      pltpu.async_copy(x_h.at[src_l.at[pl.ds(b0, BLK)]], xbuf, gsem)
      pltpu.async_copy(attr_h.at[eid_l.at[pl.ds(b0, BLK)]], abuf, asem)
      pltpu.make_async_copy(x_h.at[src_l.at[pl.ds(b0, BLK)]], xbuf, gsem).wait()
      pltpu.make_async_copy(attr_h.at[eid_l.at[pl.ds(b0, BLK)]], abuf,
                            asem).wait()

      def _acc16(g, _):
        lv = loc_l[pl.ds(b0 + g * LANES, LANES)]
        for j in range(LANES):
          loc = lv[j]
          row = g * LANES + j
          vals = [xbuf[row, pl.ds(q * LANES, LANES)]
                  for q in range(D // LANES)]
          av = abuf[row]
          for q in range(D // LANES):
            plsc.addupdate(S_acc.at[loc, pl.ds(q * LANES, LANES)], vals[q])
          plsc.addupdate(T_acc.at[loc], av)
        return 0
      lax.fori_loop(0, BLK // LANES, _acc16, 0)

      def _cix(q, _):
        lv = loc_l[pl.ds(b0 + q * LANES, LANES)]
        plsc.addupdate_scatter(cnt_v, [lv], ones_i)
        return 0
      lax.fori_loop(0, BLK // LANES, _cix, 0)

    def _chunk(ci, off):
      e0 = ci * SEG
      pltpu.sync_copy(src_h.at[pl.ds(e0, SEG)], srcb)
      pltpu.sync_copy(dst_h.at[pl.ds(e0, SEG)], dstb)

      def _scan(k, off):
        dstv = dstb[pl.ds(k * LANES, LANES)]
        srcv = srcb[pl.ds(k * LANES, LANES)]
        locv = dstv - lo
        m = jnp.logical_and(locv >= 0, locv < OWN)
        eidv = iota16 + (e0 + k * LANES)
        plsc.store_compressed(loc_l.at[pl.ds(off, LANES)], locv, mask=m)
        plsc.store_compressed(src_l.at[pl.ds(off, LANES)], srcv, mask=m)
        plsc.store_compressed(eid_l.at[pl.ds(off, LANES)], eidv, mask=m)
        return off + plsc.all_reduce_population_count(m)[0]
      off = lax.fori_loop(0, SEG // LANES, _scan, off)

      nblk = off // BLK

      def _blk(i, _):
        _drain_block(i * BLK)
        return 0
      lax.fori_loop(0, nblk, _blk, 0)

      r0 = nblk * BLK
      for q in range(BLK // LANES):
        sv = src_l[pl.ds(r0 + q * LANES, LANES)]
        lv = loc_l[pl.ds(r0 + q * LANES, LANES)]
        ev = eid_l[pl.ds(r0 + q * LANES, LANES)]
        src_l[pl.ds(q * LANES, LANES)] = sv
        loc_l[pl.ds(q * LANES, LANES)] = lv
        eid_l[pl.ds(q * LANES, LANES)] = ev
      return off - nblk * BLK
    off = lax.fori_loop(0, NSEG, _chunk, 0)

    dumpv = jnp.full((LANES,), OWN, jnp.int32)
    for q in range(BLK // LANES):
      src_l[pl.ds(off + q * LANES, LANES)] = zi
      loc_l[pl.ds(off + q * LANES, LANES)] = dumpv
      eid_l[pl.ds(off + q * LANES, LANES)] = zi
    _drain_block(0)

    pltpu.sync_copy(S_acc.at[pl.ds(0, OWN)], s_o.at[pl.ds(lo, OWN)])
    pltpu.sync_copy(T_acc.at[pl.ds(0, OWN)], t_o.at[pl.ds(lo, OWN)])
    pltpu.sync_copy(cnt_v.at[pl.ds(0, OWN)], c_o.at[0, pl.ds(lo, OWN)])

  _run_relation(x_a_h, src_ab_h, dst_ab_h, attr_ab_h, s_ab_o, t_ab_o, c_ab_o)
  _run_relation(x_b_h, src_ba_h, dst_ba_h, attr_ba_h, s_ba_o, t_ba_o, c_ba_o)


def _sc_segsums(x_a, x_b, src_ab, dst_ab, attr_ab, src_ba, dst_ba, attr_ba):
  mesh = plsc.VectorSubcoreMesh(core_axis_name="c", subcore_axis_name="s")
  f32 = jnp.float32
  out_type = (
      jax.ShapeDtypeStruct((N_ACC, D), f32),
      jax.ShapeDtypeStruct((N_ACC, D_EDGE), f32),
      jax.ShapeDtypeStruct((1, N_ACC), jnp.int32),
      jax.ShapeDtypeStruct((N_ACC, D), f32),
      jax.ShapeDtypeStruct((N_ACC, D_EDGE), f32),
      jax.ShapeDtypeStruct((1, N_ACC), jnp.int32),
  )
  scratch = [
      pltpu.VMEM((OWN + 8, D), f32),
      pltpu.VMEM((OWN + 8, D_EDGE), f32),
      pltpu.VMEM((OWN + LANES,), jnp.int32),
      pltpu.VMEM((SEG,), jnp.int32),
      pltpu.VMEM((SEG,), jnp.int32),
      pltpu.VMEM((LC,), jnp.int32),
      pltpu.VMEM((LC,), jnp.int32),
      pltpu.VMEM((LC,), jnp.int32),
      pltpu.VMEM((BLK, D), f32),
      pltpu.VMEM((BLK, D_EDGE), f32),
      pltpu.SemaphoreType.DMA,
      pltpu.SemaphoreType.DMA,
  ]
  return pl.kernel(
      _sc_body, out_type=out_type, mesh=mesh, scratch_types=scratch,
      compiler_params=pltpu.CompilerParams(
          needs_layout_passes=False, use_tc_tiling_on_sc=False),
  )(x_a, x_b, src_ab, dst_ab, attr_ab, src_ba, dst_ba, attr_ba)


BN = 2000


def _dense_body(x_ref, s_ref, t_ref, c_ref,
                wn_top, wn_bot, bn_r, ws_r, bs_r, wu_top, wu_bot, bu_r,
                wsf_r, bsf_r, out_ref):
  hi = jax.lax.Precision.HIGHEST
  x = x_ref[...]
  S = s_ref[...]
  T = t_ref[...]
  cnt = c_ref[...].astype(jnp.float32)
  summed = (jnp.dot(S, wn_top[...], precision=hi)
            + jnp.dot(T, wn_bot[...], precision=hi)
            + cnt * bn_r[...])
  agg = summed / jnp.maximum(cnt, 1.0)
  self_t = jnp.dot(x, ws_r[...], precision=hi) + bs_r[...]
  m = (jnp.dot(self_t, wu_top[...], precision=hi)
       + jnp.dot(agg, wu_bot[...], precision=hi) + bu_r[...])
  out_ref[...] = m + jnp.dot(x, wsf_r[...], precision=hi) + bsf_r[...]


def _dense(x, s_full, t_full, c_full, wn_top, wn_bot, bn, ws, bs,
           wu_top, wu_bot, bu, wsf, bsf):
  n = x.shape[0]
  grid = (n // BN,)
  row_spec = lambda width: pl.BlockSpec((BN, width), lambda i: (i, 0))
  full = lambda a: pl.BlockSpec(a.shape, lambda i: (0,) * a.ndim)
  return pl.pallas_call(
      _dense_body,
      grid=grid,
      in_specs=[
          row_spec(D), row_spec(D), row_spec(D_EDGE),
          pl.BlockSpec((BN, 1), lambda i: (i, 0)),
          full(wn_top), full(wn_bot), full(bn), full(ws), full(bs),
          full(wu_top), full(wu_bot), full(bu), full(wsf), full(bsf),
      ],
      out_specs=row_spec(D),
      out_shape=jax.ShapeDtypeStruct((n, D), jnp.float32),
  )(x, s_full, t_full, c_full,
    wn_top, wn_bot, bn, ws, bs, wu_top, wu_bot, bu, wsf, bsf)


def _pad_edges(edge_index, edge_attr):
  src = edge_index[0]
  dst = edge_index[1]
  pad = E_PAD - E_EDGES
  src = jnp.concatenate([src, jnp.zeros((pad,), jnp.int32)])
  dst = jnp.concatenate([dst, jnp.full((pad,), PAD_DST, jnp.int32)])
  attr = jnp.concatenate([edge_attr, jnp.zeros((pad, D_EDGE), jnp.float32)])
  return src, dst, attr


@jax.jit
def kernel(x_a, x_b, edge_index_ab, edge_index_ba, edge_attr_ab, edge_attr_ba,
           W_neigh_ab, b_neigh_ab, W_self_ab, b_self_ab, W_update_ab, b_update_ab,
           W_neigh_ba, b_neigh_ba, W_self_ba, b_self_ba, W_update_ba, b_update_ba,
           W_sf_a, b_sf_a, W_sf_b, b_sf_b):
  src_ab, dst_ab, attr_ab = _pad_edges(edge_index_ab, edge_attr_ab)
  src_ba, dst_ba, attr_ba = _pad_edges(edge_index_ba, edge_attr_ba)

  s_ab, t_ab, c_ab, s_ba, t_ba, c_ba = _sc_segsums(
      x_a, x_b, src_ab, dst_ab, attr_ab, src_ba, dst_ba, attr_ba)

  def two_d(b):
    return b.reshape(1, D)

  out_a = _dense(x_a, s_ba, t_ba, c_ba.reshape(N_ACC, 1),
                 W_neigh_ba[:D], W_neigh_ba[D:], two_d(b_neigh_ba),
                 W_self_ba, two_d(b_self_ba),
                 W_update_ba[:D], W_update_ba[D:], two_d(b_update_ba),
                 W_sf_a, two_d(b_sf_a))
  out_b = _dense(x_b, s_ab, t_ab, c_ab.reshape(N_ACC, 1),
                 W_neigh_ab[:D], W_neigh_ab[D:], two_d(b_neigh_ab),
                 W_self_ab, two_d(b_self_ab),
                 W_update_ab[:D], W_update_ab[D:], two_d(b_update_ab),
                 W_sf_b, two_d(b_sf_b))
  return (out_a, out_b)


# speedup vs baseline: 1.6763x; 1.4001x over previous
import jax
import jax.numpy as jnp
from jax import lax
from jax.experimental import pallas as pl
from jax.experimental.pallas import tpu as pltpu
from jax.experimental.pallas import tpu_sc as plsc

N_NODES = 10000
E_EDGES = 320000
D = 128
D_EDGE = 16

NC = 2
NS = 16
NW = NC * NS

LANES = 16
SEG = 4096
E_PAD = 327680
NSEG = E_PAD // SEG
PAD_DST = 2 ** 30
N_ACC = NW * 320
OWN = N_ACC // NW
BLK = 128
LC = SEG + 3 * BLK


def _sc_body(x_a_h, x_b_h,
             src_ab_h, dst_ab_h, attr_ab_h,
             src_ba_h, dst_ba_h, attr_ba_h,
             s_ab_o, t_ab_o, c_ab_o, s_ba_o, t_ba_o, c_ba_o,
             S_acc, T_acc, cnt_v, srcb0, dstb0, srcb1, dstb1,
             src_l, loc_l, eid_l, xbuf, abuf,
             gsem, asem, stsem0, stsem1):
  c = lax.axis_index("c")
  s = lax.axis_index("s")
  wid = s * NC + c
  lo = wid * OWN

  zf = jnp.zeros((LANES,), jnp.float32)
  zi = jnp.zeros((LANES,), jnp.int32)
  ones_i = jnp.full((LANES,), 1, jnp.int32)
  iota16 = lax.iota(jnp.int32, LANES)

  def _run_relation(x_h, src_h, dst_h, attr_h, s_o, t_o, c_o):
    def _zs(i, _):
      S_acc[i // 8, pl.ds((i % 8) * LANES, LANES)] = zf
      return 0
    lax.fori_loop(0, (OWN + 8) * 8, _zs, 0)

    def _zt(i, _):
      T_acc[i, :] = zf
      return 0
    lax.fori_loop(0, OWN + 8, _zt, 0)

    def _zc(i, _):
      cnt_v[pl.ds(i * LANES, LANES)] = zi
      return 0
    lax.fori_loop(0, (OWN + LANES) // LANES, _zc, 0)

    def _start_gathers():
      pltpu.async_copy(x_h.at[src_l.at[pl.ds(0, BLK)]], xbuf, gsem)
      pltpu.async_copy(attr_h.at[eid_l.at[pl.ds(0, BLK)]], abuf, asem)

    def _wait_gathers():
      pltpu.make_async_copy(x_h.at[src_l.at[pl.ds(0, BLK)]], xbuf, gsem).wait()
      pltpu.make_async_copy(attr_h.at[eid_l.at[pl.ds(0, BLK)]], abuf,
                            asem).wait()

    def _accumulate(off):
      def _acc16(g, _):
        lv = loc_l[pl.ds(g * LANES, LANES)]
        for j in range(LANES):
          loc = lv[j]
          row = g * LANES + j
          vals = [xbuf[row, pl.ds(q * LANES, LANES)]
                  for q in range(D // LANES)]
          av = abuf[row]
          for q in range(D // LANES):
            plsc.addupdate(S_acc.at[loc, pl.ds(q * LANES, LANES)], vals[q])
          plsc.addupdate(T_acc.at[loc], av)
        return 0
      lax.fori_loop(0, BLK // LANES, _acc16, 0)

      def _cix(q, _):
        lv = loc_l[pl.ds(q * LANES, LANES)]
        plsc.addupdate_scatter(cnt_v, [lv], ones_i)
        return 0
      lax.fori_loop(0, BLK // LANES, _cix, 0)

      def _shift(i, _):
        sv = src_l[pl.ds(BLK + i * LANES, LANES)]
        lv = loc_l[pl.ds(BLK + i * LANES, LANES)]
        ev = eid_l[pl.ds(BLK + i * LANES, LANES)]
        src_l[pl.ds(i * LANES, LANES)] = sv
        loc_l[pl.ds(i * LANES, LANES)] = lv
        eid_l[pl.ds(i * LANES, LANES)] = ev
        return 0
      lax.fori_loop(0, (off - BLK + LANES - 1) // LANES, _shift, 0)

    def _drain_step(off, pend):
      @pl.when(pend)
      def _fin():
        _wait_gathers()
        _accumulate(off)
      off = jnp.where(pend, off - BLK, off)

      nextra = jnp.maximum(off // BLK - 1, 0)

      def _extra(i, off):
        _start_gathers()
        _wait_gathers()
        _accumulate(off)
        return off - BLK
      off = lax.fori_loop(0, nextra, _extra, off)

      pend = off >= BLK

      @pl.when(pend)
      def _launch():
        _start_gathers()
      return off, pend

    def _scan_buf(srcb, dstb, e0, off):
      def _scan(k, off):
        dstv = dstb[pl.ds(k * LANES, LANES)]
        srcv = srcb[pl.ds(k * LANES, LANES)]
        locv = dstv - lo
        m = jnp.logical_and(locv >= 0, locv < OWN)
        eidv = iota16 + (e0 + k * LANES)
        plsc.store_compressed(loc_l.at[pl.ds(off, LANES)], locv, mask=m)
        plsc.store_compressed(src_l.at[pl.ds(off, LANES)], srcv, mask=m)
        plsc.store_compressed(eid_l.at[pl.ds(off, LANES)], eidv, mask=m)
        return off + plsc.all_reduce_population_count(m)[0]
      return lax.fori_loop(0, SEG // LANES, _scan, off)

    def _stage(srcb, dstb, e0, sem):
      pltpu.async_copy(src_h.at[pl.ds(e0, SEG)], srcb, sem)
      pltpu.async_copy(dst_h.at[pl.ds(e0, SEG)], dstb, sem)

    def _wait_stage(srcb, dstb, e0, sem):
      pltpu.make_async_copy(src_h.at[pl.ds(e0, SEG)], srcb, sem).wait()
      pltpu.make_async_copy(dst_h.at[pl.ds(e0, SEG)], dstb, sem).wait()

    _stage(srcb0, dstb0, 0, stsem0)

    def _super(cc, carry):
      off, pend = carry
      ea = (2 * cc) * SEG
      eb = ea + SEG
      _stage(srcb1, dstb1, eb, stsem1)
      _wait_stage(srcb0, dstb0, ea, stsem0)
      off = _scan_buf(srcb0, dstb0, ea, off)
      off, pend = _drain_step(off, pend)

      @pl.when(cc < NSEG // 2 - 1)
      def _next():
        _stage(srcb0, dstb0, ea + 2 * SEG, stsem0)
      _wait_stage(srcb1, dstb1, eb, stsem1)
      off = _scan_buf(srcb1, dstb1, eb, off)
      off, pend = _drain_step(off, pend)
      return off, pend
    off, pend = lax.fori_loop(0, NSEG // 2, _super,
                              (jnp.int32(0), jnp.bool_(False)))

    @pl.when(pend)
    def _fin_tail():
      _wait_gathers()
      _accumulate(off)
    off = jnp.where(pend, off - BLK, off)

    dumpv = jnp.full((LANES,), OWN, jnp.int32)
    for q in range(BLK // LANES):
      src_l[pl.ds(off + q * LANES, LANES)] = zi
      loc_l[pl.ds(off + q * LANES, LANES)] = dumpv
      eid_l[pl.ds(off + q * LANES, LANES)] = zi
    _start_gathers()
    _wait_gathers()
    _accumulate(jnp.int32(BLK))

    pltpu.sync_copy(S_acc.at[pl.ds(0, OWN)], s_o.at[pl.ds(lo, OWN)])
    pltpu.sync_copy(T_acc.at[pl.ds(0, OWN)], t_o.at[pl.ds(lo, OWN)])
    pltpu.sync_copy(cnt_v.at[pl.ds(0, OWN)], c_o.at[0, pl.ds(lo, OWN)])

  _run_relation(x_a_h, src_ab_h, dst_ab_h, attr_ab_h, s_ab_o, t_ab_o, c_ab_o)
  _run_relation(x_b_h, src_ba_h, dst_ba_h, attr_ba_h, s_ba_o, t_ba_o, c_ba_o)


def _sc_segsums(x_a, x_b, src_ab, dst_ab, attr_ab, src_ba, dst_ba, attr_ba):
  mesh = plsc.VectorSubcoreMesh(core_axis_name="c", subcore_axis_name="s")
  f32 = jnp.float32
  out_type = (
      jax.ShapeDtypeStruct((N_ACC, D), f32),
      jax.ShapeDtypeStruct((N_ACC, D_EDGE), f32),
      jax.ShapeDtypeStruct((1, N_ACC), jnp.int32),
      jax.ShapeDtypeStruct((N_ACC, D), f32),
      jax.ShapeDtypeStruct((N_ACC, D_EDGE), f32),
      jax.ShapeDtypeStruct((1, N_ACC), jnp.int32),
  )
  scratch = [
      pltpu.VMEM((OWN + 8, D), f32),
      pltpu.VMEM((OWN + 8, D_EDGE), f32),
      pltpu.VMEM((OWN + LANES,), jnp.int32),
      pltpu.VMEM((SEG,), jnp.int32),
      pltpu.VMEM((SEG,), jnp.int32),
      pltpu.VMEM((SEG,), jnp.int32),
      pltpu.VMEM((SEG,), jnp.int32),
      pltpu.VMEM((LC,), jnp.int32),
      pltpu.VMEM((LC,), jnp.int32),
      pltpu.VMEM((LC,), jnp.int32),
      pltpu.VMEM((BLK, D), f32),
      pltpu.VMEM((BLK, D_EDGE), f32),
      pltpu.SemaphoreType.DMA,
      pltpu.SemaphoreType.DMA,
      pltpu.SemaphoreType.DMA,
      pltpu.SemaphoreType.DMA,
  ]
  return pl.kernel(
      _sc_body, out_type=out_type, mesh=mesh, scratch_types=scratch,
      compiler_params=pltpu.CompilerParams(
          needs_layout_passes=False, use_tc_tiling_on_sc=False),
  )(x_a, x_b, src_ab, dst_ab, attr_ab, src_ba, dst_ba, attr_ba)


BN = 2000


def _dense_body(x_ref, s_ref, t_ref, c_ref,
                wn_top, wn_bot, bn_r, ws_r, bs_r, wu_top, wu_bot, bu_r,
                wsf_r, bsf_r, out_ref):
  hi = jax.lax.Precision.HIGHEST
  x = x_ref[...]
  S = s_ref[...]
  T = t_ref[...]
  cnt = c_ref[...].astype(jnp.float32)
  summed = (jnp.dot(S, wn_top[...], precision=hi)
            + jnp.dot(T, wn_bot[...], precision=hi)
            + cnt * bn_r[...])
  agg = summed / jnp.maximum(cnt, 1.0)
  self_t = jnp.dot(x, ws_r[...], precision=hi) + bs_r[...]
  m = (jnp.dot(self_t, wu_top[...], precision=hi)
       + jnp.dot(agg, wu_bot[...], precision=hi) + bu_r[...])
  out_ref[...] = m + jnp.dot(x, wsf_r[...], precision=hi) + bsf_r[...]


def _dense(x, s_full, t_full, c_full, wn_top, wn_bot, bn, ws, bs,
           wu_top, wu_bot, bu, wsf, bsf):
  n = x.shape[0]
  grid = (n // BN,)
  row_spec = lambda width: pl.BlockSpec((BN, width), lambda i: (i, 0))
  full = lambda a: pl.BlockSpec(a.shape, lambda i: (0,) * a.ndim)
  return pl.pallas_call(
      _dense_body,
      grid=grid,
      in_specs=[
          row_spec(D), row_spec(D), row_spec(D_EDGE),
          pl.BlockSpec((BN, 1), lambda i: (i, 0)),
          full(wn_top), full(wn_bot), full(bn), full(ws), full(bs),
          full(wu_top), full(wu_bot), full(bu), full(wsf), full(bsf),
      ],
      out_specs=row_spec(D),
      out_shape=jax.ShapeDtypeStruct((n, D), jnp.float32),
  )(x, s_full, t_full, c_full,
    wn_top, wn_bot, bn, ws, bs, wu_top, wu_bot, bu, wsf, bsf)


def _pad_edges(edge_index, edge_attr):
  src = edge_index[0]
  dst = edge_index[1]
  pad = E_PAD - E_EDGES
  src = jnp.concatenate([src, jnp.zeros((pad,), jnp.int32)])
  dst = jnp.concatenate([dst, jnp.full((pad,), PAD_DST, jnp.int32)])
  attr = jnp.concatenate([edge_attr, jnp.zeros((pad, D_EDGE), jnp.float32)])
  return src, dst, attr


@jax.jit
def kernel(x_a, x_b, edge_index_ab, edge_index_ba, edge_attr_ab, edge_attr_ba,
           W_neigh_ab, b_neigh_ab, W_self_ab, b_self_ab, W_update_ab, b_update_ab,
           W_neigh_ba, b_neigh_ba, W_self_ba, b_self_ba, W_update_ba, b_update_ba,
           W_sf_a, b_sf_a, W_sf_b, b_sf_b):
  src_ab, dst_ab, attr_ab = _pad_edges(edge_index_ab, edge_attr_ab)
  src_ba, dst_ba, attr_ba = _pad_edges(edge_index_ba, edge_attr_ba)

  s_ab, t_ab, c_ab, s_ba, t_ba, c_ba = _sc_segsums(
      x_a, x_b, src_ab, dst_ab, attr_ab, src_ba, dst_ba, attr_ba)

  def two_d(b):
    return b.reshape(1, D)

  out_a = _dense(x_a, s_ba, t_ba, c_ba.reshape(N_ACC, 1),
                 W_neigh_ba[:D], W_neigh_ba[D:], two_d(b_neigh_ba),
                 W_self_ba, two_d(b_self_ba),
                 W_update_ba[:D], W_update_ba[D:], two_d(b_update_ba),
                 W_sf_a, two_d(b_sf_a))
  out_b = _dense(x_b, s_ab, t_ab, c_ab.reshape(N_ACC, 1),
                 W_neigh_ab[:D], W_neigh_ab[D:], two_d(b_neigh_ab),
                 W_self_ab, two_d(b_self_ab),
                 W_update_ab[:D], W_update_ab[D:], two_d(b_update_ab),
                 W_sf_b, two_d(b_sf_b))
  return (out_a, out_b)


# speedup vs baseline: 1.9851x; 1.1842x over previous
import jax
import jax.numpy as jnp
from jax import lax
from jax.experimental import pallas as pl
from jax.experimental.pallas import tpu as pltpu
from jax.experimental.pallas import tpu_sc as plsc

N_NODES = 10000
E_EDGES = 320000
D = 128
D_EDGE = 16

NC = 2
NS = 16
NW = NC * NS

LANES = 16
SEG = 4096
E_PAD = 327680
NSEG = E_PAD // SEG
PAD_DST = 2 ** 30
N_ACC = NW * 320
OWN = N_ACC // NW
BLK = 128
LC = SEG + 3 * BLK


def _sc_body(x_a_h, x_b_h,
             src_ab_h, dst_ab_h, attr_ab_h,
             src_ba_h, dst_ba_h, attr_ba_h,
             s_ab_o, t_ab_o, c_ab_o, s_ba_o, t_ba_o, c_ba_o,
             S_acc, T_acc, cnt_v, srcb0, dstb0, srcb1, dstb1,
             src_l, loc_l, eid_l, xbuf, abuf,
             gsem, asem, stsem0, stsem1):
  c = lax.axis_index("c")
  s = lax.axis_index("s")
  wid = s * NC + c
  lo = wid * OWN

  zf = jnp.zeros((LANES,), jnp.float32)
  zi = jnp.zeros((LANES,), jnp.int32)
  ones_i = jnp.full((LANES,), 1, jnp.int32)
  iota16 = lax.iota(jnp.int32, LANES)

  def _run_relation(x_h, src_h, dst_h, attr_h, s_o, t_o, c_o):
    def _zs(i, _):
      S_acc[i // 8, pl.ds((i % 8) * LANES, LANES)] = zf
      return 0
    lax.fori_loop(0, (OWN + 8) * 8, _zs, 0)

    def _zt(i, _):
      T_acc[i, :] = zf
      return 0
    lax.fori_loop(0, OWN + 8, _zt, 0)

    def _zc(i, _):
      cnt_v[pl.ds(i * LANES, LANES)] = zi
      return 0
    lax.fori_loop(0, (OWN + LANES) // LANES, _zc, 0)

    def _start_gathers():
      pltpu.async_copy(x_h.at[src_l.at[pl.ds(0, BLK)]], xbuf, gsem)
      pltpu.async_copy(attr_h.at[eid_l.at[pl.ds(0, BLK)]], abuf, asem)

    def _wait_gathers():
      pltpu.make_async_copy(x_h.at[src_l.at[pl.ds(0, BLK)]], xbuf, gsem).wait()
      pltpu.make_async_copy(attr_h.at[eid_l.at[pl.ds(0, BLK)]], abuf,
                            asem).wait()

    def _accumulate(off):
      def _acc16(g, _):
        lv = loc_l[pl.ds(g * LANES, LANES)]
        for j in range(LANES):
          loc = lv[j]
          row = g * LANES + j
          vals = [xbuf[row, pl.ds(q * LANES, LANES)]
                  for q in range(D // LANES)]
          av = abuf[row]
          for q in range(D // LANES):
            plsc.addupdate(S_acc.at[loc, pl.ds(q * LANES, LANES)], vals[q])
          plsc.addupdate(T_acc.at[loc], av)
        return 0
      lax.fori_loop(0, BLK // LANES, _acc16, 0)

      def _cix(q, _):
        lv = loc_l[pl.ds(q * LANES, LANES)]
        plsc.addupdate_scatter(cnt_v, [lv], ones_i)
        return 0
      lax.fori_loop(0, BLK // LANES, _cix, 0)

      def _shift(i, _):
        sv = src_l[pl.ds(BLK + i * LANES, LANES)]
        lv = loc_l[pl.ds(BLK + i * LANES, LANES)]
        ev = eid_l[pl.ds(BLK + i * LANES, LANES)]
        src_l[pl.ds(i * LANES, LANES)] = sv
        loc_l[pl.ds(i * LANES, LANES)] = lv
        eid_l[pl.ds(i * LANES, LANES)] = ev
        return 0
      lax.fori_loop(0, (off - BLK + LANES - 1) // LANES, _shift, 0)

    def _drain_step(off, pend):
      @pl.when(pend)
      def _fin():
        _wait_gathers()
        _accumulate(off)
      off = jnp.where(pend, off - BLK, off)

      nextra = jnp.maximum(off // BLK - 1, 0)

      def _extra(i, off):
        _start_gathers()
        _wait_gathers()
        _accumulate(off)
        return off - BLK
      off = lax.fori_loop(0, nextra, _extra, off)

      pend = off >= BLK

      @pl.when(pend)
      def _launch():
        _start_gathers()
      return off, pend

    def _scan_buf(srcb, dstb, e0, off):
      def _scan(k, off):
        dstv = dstb[pl.ds(k * LANES, LANES)]
        srcv = srcb[pl.ds(k * LANES, LANES)]
        locv = dstv - lo
        m = jnp.logical_and(locv >= 0, locv < OWN)
        eidv = iota16 + (e0 + k * LANES)
        plsc.store_compressed(loc_l.at[pl.ds(off, LANES)], locv, mask=m)
        plsc.store_compressed(src_l.at[pl.ds(off, LANES)], srcv, mask=m)
        plsc.store_compressed(eid_l.at[pl.ds(off, LANES)], eidv, mask=m)
        return off + plsc.all_reduce_population_count(m)[0]
      return lax.fori_loop(0, SEG // LANES, _scan, off)

    def _stage(srcb, dstb, e0, sem):
      pltpu.async_copy(src_h.at[pl.ds(e0, SEG)], srcb, sem)
      pltpu.async_copy(dst_h.at[pl.ds(e0, SEG)], dstb, sem)

    def _wait_stage(srcb, dstb, e0, sem):
      pltpu.make_async_copy(src_h.at[pl.ds(e0, SEG)], srcb, sem).wait()
      pltpu.make_async_copy(dst_h.at[pl.ds(e0, SEG)], dstb, sem).wait()

    _stage(srcb0, dstb0, 0, stsem0)

    def _super(cc, carry):
      off, pend = carry
      ea = (2 * cc) * SEG
      eb = ea + SEG
      _stage(srcb1, dstb1, eb, stsem1)
      _wait_stage(srcb0, dstb0, ea, stsem0)
      off = _scan_buf(srcb0, dstb0, ea, off)
      off, pend = _drain_step(off, pend)

      @pl.when(cc < NSEG // 2 - 1)
      def _next():
        _stage(srcb0, dstb0, ea + 2 * SEG, stsem0)
      _wait_stage(srcb1, dstb1, eb, stsem1)
      off = _scan_buf(srcb1, dstb1, eb, off)
      off, pend = _drain_step(off, pend)
      return off, pend
    off, pend = lax.fori_loop(0, NSEG // 2, _super,
                              (jnp.int32(0), jnp.bool_(False)))

    @pl.when(pend)
    def _fin_tail():
      _wait_gathers()
      _accumulate(off)
    off = jnp.where(pend, off - BLK, off)

    dumpv = jnp.full((LANES,), OWN, jnp.int32)
    for q in range(BLK // LANES):
      src_l[pl.ds(off + q * LANES, LANES)] = zi
      loc_l[pl.ds(off + q * LANES, LANES)] = dumpv
      eid_l[pl.ds(off + q * LANES, LANES)] = zi
    _start_gathers()
    _wait_gathers()
    _accumulate(jnp.int32(BLK))

    pltpu.sync_copy(S_acc.at[pl.ds(0, OWN)], s_o.at[pl.ds(lo, OWN)])
    pltpu.sync_copy(T_acc.at[pl.ds(0, OWN)], t_o.at[pl.ds(lo, OWN)])
    pltpu.sync_copy(cnt_v.at[pl.ds(0, OWN)], c_o.at[0, pl.ds(lo, OWN)])

  _run_relation(x_a_h, src_ab_h, dst_ab_h, attr_ab_h, s_ab_o, t_ab_o, c_ab_o)
  _run_relation(x_b_h, src_ba_h, dst_ba_h, attr_ba_h, s_ba_o, t_ba_o, c_ba_o)


def _sc_segsums(x_a, x_b, src_ab, dst_ab, attr_ab, src_ba, dst_ba, attr_ba):
  mesh = plsc.VectorSubcoreMesh(core_axis_name="c", subcore_axis_name="s")
  f32 = jnp.float32
  out_type = (
      jax.ShapeDtypeStruct((N_ACC, D), f32),
      jax.ShapeDtypeStruct((N_ACC, D_EDGE), f32),
      jax.ShapeDtypeStruct((1, N_ACC), jnp.int32),
      jax.ShapeDtypeStruct((N_ACC, D), f32),
      jax.ShapeDtypeStruct((N_ACC, D_EDGE), f32),
      jax.ShapeDtypeStruct((1, N_ACC), jnp.int32),
  )
  scratch = [
      pltpu.VMEM((OWN + 8, D), f32),
      pltpu.VMEM((OWN + 8, D_EDGE), f32),
      pltpu.VMEM((OWN + LANES,), jnp.int32),
      pltpu.VMEM((SEG,), jnp.int32),
      pltpu.VMEM((SEG,), jnp.int32),
      pltpu.VMEM((SEG,), jnp.int32),
      pltpu.VMEM((SEG,), jnp.int32),
      pltpu.VMEM((LC,), jnp.int32),
      pltpu.VMEM((LC,), jnp.int32),
      pltpu.VMEM((LC,), jnp.int32),
      pltpu.VMEM((BLK, D), f32),
      pltpu.VMEM((BLK, D_EDGE), f32),
      pltpu.SemaphoreType.DMA,
      pltpu.SemaphoreType.DMA,
      pltpu.SemaphoreType.DMA,
      pltpu.SemaphoreType.DMA,
  ]
  return pl.kernel(
      _sc_body, out_type=out_type, mesh=mesh, scratch_types=scratch,
      compiler_params=pltpu.CompilerParams(
          needs_layout_passes=False, use_tc_tiling_on_sc=False),
  )(x_a, x_b, src_ab, dst_ab, attr_ab, src_ba, dst_ba, attr_ba)


BN = 2000


def _dense_body(x_ref, s_ref, t_ref, c_ref,
                wn_top, wn_bot, bn_r, ws_r, bs_r, wu_top, wu_bot, bu_r,
                wsf_r, bsf_r, out_ref):
  hi = jax.lax.Precision.HIGHEST
  x = x_ref[...]
  S = s_ref[...]
  T = t_ref[...]
  cnt = c_ref[...].astype(jnp.float32)
  summed = (jnp.dot(S, wn_top[...], precision=hi)
            + jnp.dot(T, wn_bot[...], precision=hi)
            + cnt * bn_r[...])
  agg = summed / jnp.maximum(cnt, 1.0)
  self_t = jnp.dot(x, ws_r[...], precision=hi) + bs_r[...]
  m = (jnp.dot(self_t, wu_top[...], precision=hi)
       + jnp.dot(agg, wu_bot[...], precision=hi) + bu_r[...])
  out_ref[...] = m + jnp.dot(x, wsf_r[...], precision=hi) + bsf_r[...]


def _dense(x, s_full, t_full, c_full, wn_top, wn_bot, bn, ws, bs,
           wu_top, wu_bot, bu, wsf, bsf):
  n = x.shape[0]
  grid = (n // BN,)
  row_spec = lambda width: pl.BlockSpec((BN, width), lambda i: (i, 0))
  full = lambda a: pl.BlockSpec(a.shape, lambda i: (0,) * a.ndim)
  return pl.pallas_call(
      _dense_body,
      grid=grid,
      in_specs=[
          row_spec(D), row_spec(D), row_spec(D_EDGE),
          pl.BlockSpec((BN, 1), lambda i: (i, 0)),
          full(wn_top), full(wn_bot), full(bn), full(ws), full(bs),
          full(wu_top), full(wu_bot), full(bu), full(wsf), full(bsf),
      ],
      out_specs=row_spec(D),
      out_shape=jax.ShapeDtypeStruct((n, D), jnp.float32),
  )(x, s_full, t_full, c_full,
    wn_top, wn_bot, bn, ws, bs, wu_top, wu_bot, bu, wsf, bsf)


def _pad_edges(edge_index):
  src = edge_index[0]
  dst = edge_index[1]
  pad = E_PAD - E_EDGES
  src = jnp.concatenate([src, jnp.zeros((pad,), jnp.int32)])
  dst = jnp.concatenate([dst, jnp.full((pad,), PAD_DST, jnp.int32)])
  return src, dst


@jax.jit
def kernel(x_a, x_b, edge_index_ab, edge_index_ba, edge_attr_ab, edge_attr_ba,
           W_neigh_ab, b_neigh_ab, W_self_ab, b_self_ab, W_update_ab, b_update_ab,
           W_neigh_ba, b_neigh_ba, W_self_ba, b_self_ba, W_update_ba, b_update_ba,
           W_sf_a, b_sf_a, W_sf_b, b_sf_b):
  src_ab, dst_ab = _pad_edges(edge_index_ab)
  src_ba, dst_ba = _pad_edges(edge_index_ba)

  s_ab, t_ab, c_ab, s_ba, t_ba, c_ba = _sc_segsums(
      x_a, x_b, src_ab, dst_ab, edge_attr_ab, src_ba, dst_ba, edge_attr_ba)

  def two_d(b):
    return b.reshape(1, D)

  out_a = _dense(x_a, s_ba, t_ba, c_ba.reshape(N_ACC, 1),
                 W_neigh_ba[:D], W_neigh_ba[D:], two_d(b_neigh_ba),
                 W_self_ba, two_d(b_self_ba),
                 W_update_ba[:D], W_update_ba[D:], two_d(b_update_ba),
                 W_sf_a, two_d(b_sf_a))
  out_b = _dense(x_b, s_ab, t_ab, c_ab.reshape(N_ACC, 1),
                 W_neigh_ab[:D], W_neigh_ab[D:], two_d(b_neigh_ab),
                 W_self_ab, two_d(b_self_ab),
                 W_update_ab[:D], W_update_ab[D:], two_d(b_update_ab),
                 W_sf_b, two_d(b_sf_b))
  return (out_a, out_b)


# speedup vs baseline: 2.5686x; 1.2939x over previous
import jax
import jax.numpy as jnp
from jax import lax
from jax.experimental import pallas as pl
from jax.experimental.pallas import tpu as pltpu
from jax.experimental.pallas import tpu_sc as plsc

N_NODES = 10000
E_EDGES = 320000
D = 128
D_EDGE = 16

NC = 2
NS = 16
NW = NC * NS

LANES = 16
SEG = 2048
E_PAD = 327680
NSEG = E_PAD // SEG
PAD_DST = 2 ** 30
N_ACC = NS * 640
OWN = N_ACC // NS
BLK = 128
LC = SEG + 3 * BLK


def _sc_body(x_a_h, x_b_h,
             src_ab_h, dst_ab_h, attr_ab_h,
             src_ba_h, dst_ba_h, attr_ba_h,
             s_ab_o, t_ab_o, c_ab_o, s_ba_o, t_ba_o, c_ba_o,
             S_acc, T_acc, cnt_v, srcb0, dstb0, srcb1, dstb1,
             src_l, loc_l, eid_l, xbuf, abuf,
             gsem, asem, stsem0, stsem1):
  c = lax.axis_index("c")
  s = lax.axis_index("s")
  lo = s * OWN

  zf = jnp.zeros((LANES,), jnp.float32)
  zi = jnp.zeros((LANES,), jnp.int32)
  ones_i = jnp.full((LANES,), 1, jnp.int32)
  iota16 = lax.iota(jnp.int32, LANES)

  def _run_relation(x_h, src_h, dst_h, attr_h, s_o, t_o, c_o):
    def _zs(i, _):
      S_acc[i // 8, pl.ds((i % 8) * LANES, LANES)] = zf
      return 0
    lax.fori_loop(0, (OWN + 8) * 8, _zs, 0)

    def _zt(i, _):
      T_acc[i, :] = zf
      return 0
    lax.fori_loop(0, OWN + 8, _zt, 0)

    def _zc(i, _):
      cnt_v[pl.ds(i * LANES, LANES)] = zi
      return 0
    lax.fori_loop(0, (OWN + LANES) // LANES, _zc, 0)

    def _start_gathers():
      pltpu.async_copy(x_h.at[src_l.at[pl.ds(0, BLK)]], xbuf, gsem)
      pltpu.async_copy(attr_h.at[eid_l.at[pl.ds(0, BLK)]], abuf, asem)

    def _wait_gathers():
      pltpu.make_async_copy(x_h.at[src_l.at[pl.ds(0, BLK)]], xbuf, gsem).wait()
      pltpu.make_async_copy(attr_h.at[eid_l.at[pl.ds(0, BLK)]], abuf,
                            asem).wait()

    def _accumulate(off):
      def _acc16(g, _):
        lv = loc_l[pl.ds(g * LANES, LANES)]
        for j in range(LANES):
          loc = lv[j]
          row = g * LANES + j
          vals = [xbuf[row, pl.ds(q * LANES, LANES)]
                  for q in range(D // LANES)]
          av = abuf[row]
          for q in range(D // LANES):
            plsc.addupdate(S_acc.at[loc, pl.ds(q * LANES, LANES)], vals[q])
          plsc.addupdate(T_acc.at[loc], av)
        return 0
      lax.fori_loop(0, BLK // LANES, _acc16, 0)

      def _cix(q, _):
        lv = loc_l[pl.ds(q * LANES, LANES)]
        plsc.addupdate_scatter(cnt_v, [lv], ones_i)
        return 0
      lax.fori_loop(0, BLK // LANES, _cix, 0)

      def _shift(i, _):
        sv = src_l[pl.ds(BLK + i * LANES, LANES)]
        lv = loc_l[pl.ds(BLK + i * LANES, LANES)]
        ev = eid_l[pl.ds(BLK + i * LANES, LANES)]
        src_l[pl.ds(i * LANES, LANES)] = sv
        loc_l[pl.ds(i * LANES, LANES)] = lv
        eid_l[pl.ds(i * LANES, LANES)] = ev
        return 0
      lax.fori_loop(0, (off - BLK + LANES - 1) // LANES, _shift, 0)

    def _drain_step(off, pend):
      @pl.when(pend)
      def _fin():
        _wait_gathers()
        _accumulate(off)
      off = jnp.where(pend, off - BLK, off)

      nextra = jnp.maximum(off // BLK - 1, 0)

      def _extra(i, off):
        _start_gathers()
        _wait_gathers()
        _accumulate(off)
        return off - BLK
      off = lax.fori_loop(0, nextra, _extra, off)

      pend = off >= BLK

      @pl.when(pend)
      def _launch():
        _start_gathers()
      return off, pend

    def _scan_buf(srcb, dstb, e0, off):
      def _scan(k, off):
        dstv = dstb[pl.ds(k * LANES, LANES)]
        srcv = srcb[pl.ds(k * LANES, LANES)]
        locv = dstv - lo
        m = jnp.logical_and(locv >= 0, locv < OWN)
        eidv = iota16 + (e0 + k * LANES)
        plsc.store_compressed(loc_l.at[pl.ds(off, LANES)], locv, mask=m)
        plsc.store_compressed(src_l.at[pl.ds(off, LANES)], srcv, mask=m)
        plsc.store_compressed(eid_l.at[pl.ds(off, LANES)], eidv, mask=m)
        return off + plsc.all_reduce_population_count(m)[0]
      return lax.fori_loop(0, SEG // LANES, _scan, off)

    def _stage(srcb, dstb, e0, sem):
      pltpu.async_copy(src_h.at[pl.ds(e0, SEG)], srcb, sem)
      pltpu.async_copy(dst_h.at[pl.ds(e0, SEG)], dstb, sem)

    def _wait_stage(srcb, dstb, e0, sem):
      pltpu.make_async_copy(src_h.at[pl.ds(e0, SEG)], srcb, sem).wait()
      pltpu.make_async_copy(dst_h.at[pl.ds(e0, SEG)], dstb, sem).wait()

    _stage(srcb0, dstb0, 0, stsem0)

    def _super(cc, carry):
      off, pend = carry
      ea = (2 * cc) * SEG
      eb = ea + SEG
      _stage(srcb1, dstb1, eb, stsem1)
      _wait_stage(srcb0, dstb0, ea, stsem0)
      off = _scan_buf(srcb0, dstb0, ea, off)
      off, pend = _drain_step(off, pend)

      @pl.when(cc < NSEG // 2 - 1)
      def _next():
        _stage(srcb0, dstb0, ea + 2 * SEG, stsem0)
      _wait_stage(srcb1, dstb1, eb, stsem1)
      off = _scan_buf(srcb1, dstb1, eb, off)
      off, pend = _drain_step(off, pend)
      return off, pend
    off, pend = lax.fori_loop(0, NSEG // 2, _super,
                              (jnp.int32(0), jnp.bool_(False)))

    @pl.when(pend)
    def _fin_tail():
      _wait_gathers()
      _accumulate(off)
    off = jnp.where(pend, off - BLK, off)

    dumpv = jnp.full((LANES,), OWN, jnp.int32)
    for q in range(BLK // LANES):
      src_l[pl.ds(off + q * LANES, LANES)] = zi
      loc_l[pl.ds(off + q * LANES, LANES)] = dumpv
      eid_l[pl.ds(off + q * LANES, LANES)] = zi
    _start_gathers()
    _wait_gathers()
    _accumulate(jnp.int32(BLK))

    pltpu.sync_copy(S_acc.at[pl.ds(0, OWN)], s_o.at[pl.ds(lo, OWN)])
    pltpu.sync_copy(T_acc.at[pl.ds(0, OWN)], t_o.at[pl.ds(lo, OWN)])
    pltpu.sync_copy(cnt_v.at[pl.ds(0, OWN)], c_o.at[0, pl.ds(lo, OWN)])

  @pl.when(c == 0)
  def _rel_ab():
    _run_relation(x_a_h, src_ab_h, dst_ab_h, attr_ab_h, s_ab_o, t_ab_o, c_ab_o)

  @pl.when(c == 1)
  def _rel_ba():
    _run_relation(x_b_h, src_ba_h, dst_ba_h, attr_ba_h, s_ba_o, t_ba_o, c_ba_o)


def _sc_segsums(x_a, x_b, src_ab, dst_ab, attr_ab, src_ba, dst_ba, attr_ba):
  mesh = plsc.VectorSubcoreMesh(core_axis_name="c", subcore_axis_name="s")
  f32 = jnp.float32
  out_type = (
      jax.ShapeDtypeStruct((N_ACC, D), f32),
      jax.ShapeDtypeStruct((N_ACC, D_EDGE), f32),
      jax.ShapeDtypeStruct((1, N_ACC), jnp.int32),
      jax.ShapeDtypeStruct((N_ACC, D), f32),
      jax.ShapeDtypeStruct((N_ACC, D_EDGE), f32),
      jax.ShapeDtypeStruct((1, N_ACC), jnp.int32),
  )
  scratch = [
      pltpu.VMEM((OWN + 8, D), f32),
      pltpu.VMEM((OWN + 8, D_EDGE), f32),
      pltpu.VMEM((OWN + LANES,), jnp.int32),
      pltpu.VMEM((SEG,), jnp.int32),
      pltpu.VMEM((SEG,), jnp.int32),
      pltpu.VMEM((SEG,), jnp.int32),
      pltpu.VMEM((SEG,), jnp.int32),
      pltpu.VMEM((LC,), jnp.int32),
      pltpu.VMEM((LC,), jnp.int32),
      pltpu.VMEM((LC,), jnp.int32),
      pltpu.VMEM((BLK, D), f32),
      pltpu.VMEM((BLK, D_EDGE), f32),
      pltpu.SemaphoreType.DMA,
      pltpu.SemaphoreType.DMA,
      pltpu.SemaphoreType.DMA,
      pltpu.SemaphoreType.DMA,
  ]
  return pl.kernel(
      _sc_body, out_type=out_type, mesh=mesh, scratch_types=scratch,
      compiler_params=pltpu.CompilerParams(
          needs_layout_passes=False, use_tc_tiling_on_sc=False),
  )(x_a, x_b, src_ab, dst_ab, attr_ab, src_ba, dst_ba, attr_ba)


BN = 2000


def _dense_body(x_ref, s_ref, t_ref, c_ref,
                wn_top, wn_bot, bn_r, ws_r, bs_r, wu_top, wu_bot, bu_r,
                wsf_r, bsf_r, out_ref):
  hi = jax.lax.Precision.HIGHEST
  x = x_ref[...]
  S = s_ref[...]
  T = t_ref[...]
  cnt = c_ref[...].astype(jnp.float32)
  summed = (jnp.dot(S, wn_top[...], precision=hi)
            + jnp.dot(T, wn_bot[...], precision=hi)
            + cnt * bn_r[...])
  agg = summed / jnp.maximum(cnt, 1.0)
  self_t = jnp.dot(x, ws_r[...], precision=hi) + bs_r[...]
  m = (jnp.dot(self_t, wu_top[...], precision=hi)
       + jnp.dot(agg, wu_bot[...], precision=hi) + bu_r[...])
  out_ref[...] = m + jnp.dot(x, wsf_r[...], precision=hi) + bsf_r[...]


def _dense(x, s_full, t_full, c_full, wn_top, wn_bot, bn, ws, bs,
           wu_top, wu_bot, bu, wsf, bsf):
  n = x.shape[0]
  grid = (n // BN,)
  row_spec = lambda width: pl.BlockSpec((BN, width), lambda i: (i, 0))
  full = lambda a: pl.BlockSpec(a.shape, lambda i: (0,) * a.ndim)
  return pl.pallas_call(
      _dense_body,
      grid=grid,
      in_specs=[
          row_spec(D), row_spec(D), row_spec(D_EDGE),
          pl.BlockSpec((BN, 1), lambda i: (i, 0)),
          full(wn_top), full(wn_bot), full(bn), full(ws), full(bs),
          full(wu_top), full(wu_bot), full(bu), full(wsf), full(bsf),
      ],
      out_specs=row_spec(D),
      out_shape=jax.ShapeDtypeStruct((n, D), jnp.float32),
  )(x, s_full, t_full, c_full,
    wn_top, wn_bot, bn, ws, bs, wu_top, wu_bot, bu, wsf, bsf)


def _pad_edges(edge_index):
  src = edge_index[0]
  dst = edge_index[1]
  pad = E_PAD - E_EDGES
  src = jnp.concatenate([src, jnp.zeros((pad,), jnp.int32)])
  dst = jnp.concatenate([dst, jnp.full((pad,), PAD_DST, jnp.int32)])
  return src, dst


@jax.jit
def kernel(x_a, x_b, edge_index_ab, edge_index_ba, edge_attr_ab, edge_attr_ba,
           W_neigh_ab, b_neigh_ab, W_self_ab, b_self_ab, W_update_ab, b_update_ab,
           W_neigh_ba, b_neigh_ba, W_self_ba, b_self_ba, W_update_ba, b_update_ba,
           W_sf_a, b_sf_a, W_sf_b, b_sf_b):
  src_ab, dst_ab = _pad_edges(edge_index_ab)
  src_ba, dst_ba = _pad_edges(edge_index_ba)

  s_ab, t_ab, c_ab, s_ba, t_ba, c_ba = _sc_segsums(
      x_a, x_b, src_ab, dst_ab, edge_attr_ab, src_ba, dst_ba, edge_attr_ba)

  def two_d(b):
    return b.reshape(1, D)

  out_a = _dense(x_a, s_ba, t_ba, c_ba.reshape(N_ACC, 1),
                 W_neigh_ba[:D], W_neigh_ba[D:], two_d(b_neigh_ba),
                 W_self_ba, two_d(b_self_ba),
                 W_update_ba[:D], W_update_ba[D:], two_d(b_update_ba),
                 W_sf_a, two_d(b_sf_a))
  out_b = _dense(x_b, s_ab, t_ab, c_ab.reshape(N_ACC, 1),
                 W_neigh_ab[:D], W_neigh_ab[D:], two_d(b_neigh_ab),
                 W_self_ab, two_d(b_self_ab),
                 W_update_ab[:D], W_update_ab[D:], two_d(b_update_ab),
                 W_sf_b, two_d(b_sf_b))
  return (out_a, out_b)
